# Initial kernel scaffold; baseline (speedup 1.0000x reference)
import functools

import jax
import jax.numpy as jnp
from jax import lax
from jax.experimental import pallas as pl
from jax.experimental.pallas import tpu as pltpu
from jax.experimental.pallas import tpu_sc as plsc

R = 16
N = 100000
B = 16384
REG_PARA = 0.01
NC, NS, L = 2, 16, 16
NW = NC * NS
PER_W = B // NW
C = 64
NCH = PER_W // C
KSTREAM = 128


def _sc_body(ix0, ix1, ix2, w0, p0, w1, p1, w2, p2, out, partials,
             ixb0, ixb1, ixb2, idx1, idx2,
             r0w, r0p, r1w, r1p, r2w, r2p,
             v0buf, obuf, regbuf, sem):
    wid = lax.axis_index("c") * NS + lax.axis_index("s")
    base = wid * PER_W

    pltpu.sync_copy(ix0.at[pl.ds(base, PER_W)], ixb0)
    pltpu.sync_copy(ix1.at[pl.ds(base, PER_W)], ixb1)
    pltpu.sync_copy(ix2.at[pl.ds(base, PER_W)], ixb2)

    s0 = jnp.zeros((L,), jnp.float32)
    s1 = jnp.zeros((L,), jnp.float32)
    s2 = jnp.zeros((L,), jnp.float32)

    ramp = lax.iota(jnp.int32, L) * N

    for j in range(NCH):
        o = j * C

        def build1(r, carry):
            for bb in range(C // L):
                idx1[pl.ds(r * C + bb * L, L)] = (
                    ixb1[pl.ds(o + bb * L, L)] + r * N
                )
            return carry
        lax.fori_loop(0, R, build1, 0)

        def build2(b, carry):
            idx2[pl.ds(b * L, L)] = ixb2[o + b] + ramp
            return carry
        lax.fori_loop(0, C, build2, 0)

        copies = []
        copies.append(pltpu.async_copy(w0.at[ixb0.at[pl.ds(o, C)]], r0w, sem))
        copies.append(pltpu.async_copy(p0.at[ixb0.at[pl.ds(o, C)]], r0p, sem))
        for k in range(C * R // KSTREAM):
            sl = pl.ds(k * KSTREAM, KSTREAM)
            copies.append(pltpu.async_copy(w1.at[idx1.at[sl]], r1w.at[sl], sem))
            copies.append(pltpu.async_copy(p1.at[idx1.at[sl]], r1p.at[sl], sem))
            copies.append(pltpu.async_copy(w2.at[idx2.at[sl]], r2w.at[sl], sem))
            copies.append(pltpu.async_copy(p2.at[idx2.at[sl]], r2p.at[sl], sem))
        for cp in copies:
            cp.wait()

        def elem(b, carry):
            s0, s1, s2 = carry
            v0 = r0w[b, :] * r0p[b, :]
            v2 = r2w[pl.ds(b * L, L)] * r2p[pl.ds(b * L, L)]
            v0buf[:] = v0
            t = jnp.zeros((L,), jnp.float32)
            msum = jnp.zeros((L,), jnp.float32)
            for r in range(R):
                m = r1w[r * C + b, :] * r1p[r * C + b, :]
                msum = msum + m
                t = t + v0buf[r] * m
            obuf[b] = jnp.sum(t * v2)
            return (s0 + v0, s1 + msum, s2 + v2)

        s0, s1, s2 = lax.fori_loop(0, C, elem, (s0, s1, s2))
        pltpu.sync_copy(obuf, out.at[pl.ds(base + o, C)])

    regbuf[0, :] = s0
    regbuf[1, :] = s1
    regbuf[2, :] = s2
    pltpu.sync_copy(regbuf, partials.at[wid])


@jax.jit
def _tt_lookup(ix0, ix1, ix2, w0, p0, w1, p1, w2, p2):
    mesh = plsc.VectorSubcoreMesh(core_axis_name="c", subcore_axis_name="s")
    f = pl.kernel(
        _sc_body,
        out_type=[
            jax.ShapeDtypeStruct((B,), jnp.float32),
            jax.ShapeDtypeStruct((NW, 3, L), jnp.float32),
        ],
        mesh=mesh,
        scratch_types=[
            pltpu.VMEM((PER_W,), jnp.int32),
            pltpu.VMEM((PER_W,), jnp.int32),
            pltpu.VMEM((PER_W,), jnp.int32),
            pltpu.VMEM((C * R,), jnp.int32),
            pltpu.VMEM((C * R,), jnp.int32),
            pltpu.VMEM((C, R), jnp.float32),
            pltpu.VMEM((C, R), jnp.float32),
            pltpu.VMEM((C * R, R), jnp.float32),
            pltpu.VMEM((C * R, R), jnp.float32),
            pltpu.VMEM((C * R,), jnp.float32),
            pltpu.VMEM((C * R,), jnp.float32),
            pltpu.VMEM((L,), jnp.float32),
            pltpu.VMEM((C,), jnp.float32),
            pltpu.VMEM((3, L), jnp.float32),
            pltpu.SemaphoreType.DMA,
        ],
    )
    return f(ix0, ix1, ix2, w0, p0, w1, p1, w2, p2)


def kernel(indices, W0, W1, W2, P0, P1, P2):
    ix0 = indices[:, 0]
    ix1 = indices[:, 1]
    ix2 = indices[:, 2]
    w0 = W0.reshape(N, R)
    p0 = P0.reshape(N, R)
    w1 = W1.reshape(R * N, R)
    p1 = P1.reshape(R * N, R)
    w2 = W2.reshape(R * N)
    p2 = P2.reshape(R * N)
    preds, partials = _tt_lookup(ix0, ix1, ix2, w0, p0, w1, p1, w2, p2)
    s = jnp.sum(partials, axis=(0, 2))
    reg = REG_PARA * (jnp.abs(s[0]) / (B * R)
                      + jnp.abs(s[1]) / (B * R * R)
                      + jnp.abs(s[2]) / (B * R))
    return preds, reg


# speedup vs baseline: 2.2613x; 2.2613x over previous
import functools

import jax
import jax.numpy as jnp
from jax import lax
from jax.experimental import pallas as pl
from jax.experimental.pallas import tpu as pltpu
from jax.experimental.pallas import tpu_sc as plsc

R = 16
N = 100000
B = 16384
REG_PARA = 0.01
NC, NS, L = 2, 16, 16
NW = NC * NS
PER_W = B // NW
C = 64
NCH = PER_W // C
KSTREAM = 128


def _bcast(v, i):
    return v[i]


def _sc_body(ix0, ix1, ix2, w0, p0, w1, p1, w2, p2, out, partials,
             ixb0, ixb1, ixb2, idx1, idx2,
             r0w, r0p, r1w, r1p, r2w, r2p,
             obuf, regbuf, sem):
    wid = lax.axis_index("c") * NS + lax.axis_index("s")
    base = wid * PER_W

    pltpu.sync_copy(ix0.at[pl.ds(base, PER_W)], ixb0)
    pltpu.sync_copy(ix1.at[pl.ds(base, PER_W)], ixb1)
    pltpu.sync_copy(ix2.at[pl.ds(base, PER_W)], ixb2)

    zeros = jnp.zeros((L,), jnp.float32)
    ramp = lax.iota(jnp.int32, L) * N
    lanes = lax.iota(jnp.int32, L)

    def subchunk(j, carry):
        s0, s1, s2 = carry
        o = j * C

        def build1(r, c2):
            for bb in range(C // L):
                idx1[pl.ds(r * C + bb * L, L)] = (
                    ixb1[pl.ds(o + bb * L, L)] + r * N
                )
            return c2
        lax.fori_loop(0, R, build1, 0)

        def build2(g, c2):
            vex = ixb2[pl.ds(o + g * L, L)]
            for i in range(L):
                idx2[pl.ds((g * L + i) * L, L)] = _bcast(vex, i) + ramp
            return c2
        lax.fori_loop(0, C // L, build2, 0)

        copies = []
        copies.append(pltpu.async_copy(w0.at[ixb0.at[pl.ds(o, C)]], r0w, sem))
        copies.append(pltpu.async_copy(p0.at[ixb0.at[pl.ds(o, C)]], r0p, sem))
        for k in range(C * R // KSTREAM):
            sl = pl.ds(k * KSTREAM, KSTREAM)
            copies.append(pltpu.async_copy(w1.at[idx1.at[sl]], r1w.at[sl], sem))
            copies.append(pltpu.async_copy(p1.at[idx1.at[sl]], r1p.at[sl], sem))
            copies.append(pltpu.async_copy(w2.at[idx2.at[sl]], r2w.at[sl], sem))
            copies.append(pltpu.async_copy(p2.at[idx2.at[sl]], r2p.at[sl], sem))
        for cp in copies:
            cp.wait()

        def group(g, c2):
            s0, s1, s2 = c2
            outv = zeros
            for i in range(L):
                b = g * L + i
                v0 = r0w[b, :] * r0p[b, :]
                v2 = r2w[pl.ds(b * L, L)] * r2p[pl.ds(b * L, L)]
                t = zeros
                msum = zeros
                for r in range(R):
                    m = r1w[r * C + b, :] * r1p[r * C + b, :]
                    msum = msum + m
                    t = t + _bcast(v0, r) * m
                sval = jnp.sum(t * v2)
                outv = jnp.where(lanes == i, sval, outv)
                s0 = s0 + v0
                s1 = s1 + msum
                s2 = s2 + v2
            obuf[pl.ds(g * L, L)] = outv
            return (s0, s1, s2)

        s0, s1, s2 = lax.fori_loop(0, C // L, group, (s0, s1, s2))
        pltpu.sync_copy(obuf, out.at[pl.ds(base + o, C)])
        return (s0, s1, s2)

    s0, s1, s2 = lax.fori_loop(0, NCH, subchunk, (zeros, zeros, zeros))

    regbuf[0, :] = s0
    regbuf[1, :] = s1
    regbuf[2, :] = s2
    pltpu.sync_copy(regbuf, partials.at[wid])


@jax.jit
def _tt_lookup(ix0, ix1, ix2, w0, p0, w1, p1, w2, p2):
    mesh = plsc.VectorSubcoreMesh(core_axis_name="c", subcore_axis_name="s")
    f = pl.kernel(
        _sc_body,
        out_type=[
            jax.ShapeDtypeStruct((B,), jnp.float32),
            jax.ShapeDtypeStruct((NW, 3, L), jnp.float32),
        ],
        mesh=mesh,
        compiler_params=pltpu.CompilerParams(
            needs_layout_passes=False, use_tc_tiling_on_sc=False),
        scratch_types=[
            pltpu.VMEM((PER_W,), jnp.int32),
            pltpu.VMEM((PER_W,), jnp.int32),
            pltpu.VMEM((PER_W,), jnp.int32),
            pltpu.VMEM((C * R,), jnp.int32),
            pltpu.VMEM((C * R,), jnp.int32),
            pltpu.VMEM((C, R), jnp.float32),
            pltpu.VMEM((C, R), jnp.float32),
            pltpu.VMEM((C * R, R), jnp.float32),
            pltpu.VMEM((C * R, R), jnp.float32),
            pltpu.VMEM((C * R,), jnp.float32),
            pltpu.VMEM((C * R,), jnp.float32),
            pltpu.VMEM((C,), jnp.float32),
            pltpu.VMEM((3, L), jnp.float32),
            pltpu.SemaphoreType.DMA,
        ],
    )
    return f(ix0, ix1, ix2, w0, p0, w1, p1, w2, p2)


def kernel(indices, W0, W1, W2, P0, P1, P2):
    ix0 = indices[:, 0]
    ix1 = indices[:, 1]
    ix2 = indices[:, 2]
    w0 = W0.reshape(N, R)
    p0 = P0.reshape(N, R)
    w1 = W1.reshape(R * N, R)
    p1 = P1.reshape(R * N, R)
    w2 = W2.reshape(R * N)
    p2 = P2.reshape(R * N)
    preds, partials = _tt_lookup(ix0, ix1, ix2, w0, p0, w1, p1, w2, p2)
    s = jnp.sum(partials, axis=(0, 2))
    reg = REG_PARA * (jnp.abs(s[0]) / (B * R)
                      + jnp.abs(s[1]) / (B * R * R)
                      + jnp.abs(s[2]) / (B * R))
    return preds, reg
